# baseline (speedup 1.0000x reference)
import functools
import math

import jax
import jax.numpy as jnp
from jax.experimental import pallas as pl
from jax.experimental.pallas import tpu as pltpu


def _full_spec(shape):
    zeros = (0,) * len(shape)
    return pl.BlockSpec(tuple(shape), lambda b: zeros)


def _encoder_kernel(x_ref, aw_ref, au_ref,
                    w1_ref, b1_ref, w2_ref, b2_ref, w3_ref, b3_ref,
                    whh1_ref, pb1_ref, whh2_ref, pb2_ref, whh3_ref, pb3_ref,
                    out_ref, *, g, n0, k1, k2, k3):
    f32 = jnp.float32

    def level(aw2, au2, xs, w_ref, b_ref, whh_ref, brel_ref,
              n, k, want_adj):
        w = w_ref[...]
        b_row = b_ref[...]
        xw = jnp.dot(xs, w, preferred_element_type=f32)
        d = jax.lax.rsqrt(jnp.sum(aw2, axis=1, keepdims=True) + 1.0)
        dxw = d * xw
        agg = jnp.concatenate(
            [jnp.dot(aw2[i * n:(i + 1) * n, :], dxw[i * n:(i + 1) * n, :],
                     preferred_element_type=f32) for i in range(g)], axis=0)
        h = jnp.maximum(d * agg + (d * d) * xw + b_row, 0.0)

        hwr = jnp.dot(h, whh_ref[...], preferred_element_type=f32)
        hw = hwr[:, 0:1]
        brel = brel_ref[...]
        s = jnp.concatenate(
            [jnp.tanh(jnp.dot(au2[i * n:(i + 1) * n, :],
                              hw[i * n:(i + 1) * n, :],
                              preferred_element_type=f32)
                      + hwr[i * n:(i + 1) * n, 1:2] + brel)
             for i in range(g)], axis=0)
        hs = h * s

        sub = jax.lax.broadcasted_iota(jnp.int32, (n, 1), 0)
        lane = jax.lax.broadcasted_iota(jnp.int32, (1, n), 1)
        eye = sub == lane
        ksub = jax.lax.broadcasted_iota(jnp.int32, (k, 1), 0)
        nh = xw.shape[1]
        new_aw, new_au, xs_next, ros = [], [], [], []
        for i in range(g):
            si = s[i * n:(i + 1) * n, :]
            s_row = jnp.sum(jnp.where(eye, si, 0.0), axis=0,
                            keepdims=True)
            dom = (si > s_row) | ((si == s_row) & (sub < lane))
            rank = jnp.sum(dom.astype(jnp.int32), axis=0,
                           keepdims=True)
            p = (rank == ksub).astype(f32)
            if want_adj:
                cat = jnp.concatenate(
                    [hs[i * n:(i + 1) * n, :], aw2[i * n:(i + 1) * n, :],
                     au2[i * n:(i + 1) * n, :]], axis=1)
                pooled = jnp.dot(p, cat, preferred_element_type=f32)
                xp = pooled[:, :nh]
                both = jax.lax.dot_general(
                    jnp.concatenate([pooled[:, nh:nh + n],
                                     pooled[:, nh + n:]], axis=0), p,
                    (((1,), (1,)), ((), ())),
                    preferred_element_type=f32)
                new_aw.append(both[:k, :])
                new_au.append(both[k:, :])
            else:
                xp = jnp.dot(p, hs[i * n:(i + 1) * n, :],
                             preferred_element_type=f32)
            xs_next.append(xp)
            ros.append((jnp.max(xp, axis=0, keepdims=True),
                        jnp.mean(xp, axis=0, keepdims=True)))
        naw = jnp.concatenate(new_aw, axis=0) if want_adj else None
        nau = jnp.concatenate(new_au, axis=0) if want_adj else None
        return naw, nau, jnp.concatenate(xs_next, axis=0), ros

    aw1 = aw_ref[...].reshape(g * n0, n0)
    au1 = au_ref[...].reshape(g * n0, n0)
    x0 = x_ref[...].reshape(g * n0, x_ref.shape[2])

    aw2, au2, xp1, ro1 = level(aw1, au1, x0, w1_ref, b1_ref,
                               whh1_ref, pb1_ref, n0, k1, True)
    aw3, au3, xp2, ro2 = level(aw2, au2, xp1, w2_ref, b2_ref,
                               whh2_ref, pb2_ref, k1, k2, True)
    _, _, _, ro3 = level(aw3, au3, xp2, w3_ref, b3_ref,
                         whh3_ref, pb3_ref, k2, k3, False)

    rows = [jnp.concatenate([ro1[i][0], ro1[i][1], ro2[i][0], ro2[i][1],
                             ro3[i][0], ro3[i][1]], axis=1)
            for i in range(g)]
    out_ref[...] = jnp.concatenate(rows, axis=0)


def _encoder(x_all, aw_all, au_all, conv_ws, conv_bs, pool_ws, pool_bs, ratio):
    mb, n0, f = x_all.shape
    nh = conv_ws[0].shape[1]
    k1 = int(math.ceil(ratio * n0))
    k2 = int(math.ceil(ratio * k1))
    k3 = int(math.ceil(ratio * k2))
    g = 64 if mb % 64 == 0 else 1

    w1, w2, w3 = conv_ws
    b1, b2, b3 = (b.reshape(1, nh) for b in conv_bs)
    pr1, po1, pr2, po2, pr3, po3 = pool_ws
    whh1 = jnp.concatenate([pr1, po1], axis=1)
    whh2 = jnp.concatenate([pr2, po2], axis=1)
    whh3 = jnp.concatenate([pr3, po3], axis=1)
    pb1, pb2, pb3 = (b.reshape(1, 1) for b in pool_bs)

    out = pl.pallas_call(
        functools.partial(_encoder_kernel, g=g, n0=n0, k1=k1, k2=k2, k3=k3),
        grid=(mb // g,),
        in_specs=[
            pl.BlockSpec((g, n0, f), lambda b: (b, 0, 0)),
            pl.BlockSpec((g, n0, n0), lambda b: (b, 0, 0)),
            pl.BlockSpec((g, n0, n0), lambda b: (b, 0, 0)),
            _full_spec((f, nh)), _full_spec((1, nh)),
            _full_spec((nh, nh)), _full_spec((1, nh)),
            _full_spec((nh, nh)), _full_spec((1, nh)),
            _full_spec((nh, 2)), _full_spec((1, 1)),
            _full_spec((nh, 2)), _full_spec((1, 1)),
            _full_spec((nh, 2)), _full_spec((1, 1)),
        ],
        out_specs=pl.BlockSpec((g, 6 * nh), lambda b: (b, 0)),
        out_shape=jax.ShapeDtypeStruct((mb, 6 * nh), jnp.float32),
        compiler_params=pltpu.CompilerParams(
            dimension_semantics=("parallel",)),
    )(x_all, aw_all, au_all, w1, b1, w2, b2, w3, b3,
      whh1, pb1, whh2, pb2, whh3, pb3)
    return out


def _ddi_kernel(feat_ref, attr_ref, nnw_ref, nnb_ref,
                src_ref, tgt_ref, asrc_ref, atgt_ref,
                root_w_ref, root_b_ref, lin1_w_ref, lin1_b_ref,
                lin2_w_ref, lin2_b_ref,
                loss_ref, score_ref, posx_ref, *, e_pos, e_neg, n_nodes, n_chan):
    f32 = jnp.float32
    feat = feat_ref[...]
    attr = attr_ref[...]
    src = src_ref[...]
    tgt = tgt_ref[...]
    asrc = asrc_ref[...]
    atgt = atgt_ref[...]

    node_lane = jax.lax.broadcasted_iota(jnp.int32, (1, n_nodes), 1)
    node_sub = jax.lax.broadcasted_iota(jnp.int32, (n_nodes, 1), 0)

    gsrc = (src == node_lane).astype(f32)
    xs = jnp.dot(gsrc, feat, preferred_element_type=f32)

    msg = jnp.dot(xs, nnb_ref[...], preferred_element_type=f32)
    for c in range(n_chan):
        msg = msg + attr[:, c:c + 1] * jnp.dot(xs, nnw_ref[c],
                                               preferred_element_type=f32)

    scat = (node_sub == tgt).astype(f32)
    agg = jnp.dot(scat, msg, preferred_element_type=f32)

    x_ddi = jnp.maximum(jnp.dot(feat, root_w_ref[...],
                                preferred_element_type=f32)
                        + agg + root_b_ref[...], 0.0)

    gsrc_all = (asrc == node_lane).astype(f32)
    gtgt_all = (atgt == node_lane).astype(f32)
    src_feat = jnp.dot(gsrc_all, x_ddi, preferred_element_type=f32)
    tgt_feat = jnp.dot(gtgt_all, x_ddi, preferred_element_type=f32)
    fx = jnp.dot(src_feat, lin1_w_ref[...],
                 preferred_element_type=f32) + lin1_b_ref[...]
    fy = jnp.dot(tgt_feat, lin2_w_ref[...],
                 preferred_element_type=f32) + lin2_b_ref[...]

    scores = jnp.sum(fx * fy, axis=1, keepdims=True)
    score_ref[...] = scores
    posx_ref[...] = fx[:e_pos, :]

    def sp(v):
        return jnp.maximum(v, 0.0) + jnp.log(1.0 + jnp.exp(-jnp.abs(v)))

    row = jax.lax.broadcasted_iota(jnp.int32, scores.shape, 0)
    is_pos = row < e_pos
    loss_pos = jnp.sum(jnp.where(is_pos, sp(-scores), 0.0),
                       axis=0, keepdims=True) / e_pos
    loss_neg = jnp.sum(jnp.where(is_pos, 0.0, sp(scores)),
                       axis=0, keepdims=True) / e_neg
    loss_ref[...] = loss_pos + loss_neg


def _vmem():
    return pl.BlockSpec(memory_space=pltpu.MemorySpace.VMEM)


def _ddi_head(feat, attr, nnw3, nnb2, src, tgt_row, src_all, tgt_all,
              root_w, root_b, lin1_w, lin1_b, lin2_w, lin2_b,
              *, e_pos, e_neg):
    m, _ = feat.shape
    fout = root_w.shape[1]
    c = attr.shape[1]
    e_all = src_all.shape[0]
    return pl.pallas_call(
        functools.partial(_ddi_kernel, e_pos=e_pos, e_neg=e_neg,
                          n_nodes=m, n_chan=c),
        in_specs=[_vmem()] * 14,
        out_specs=[_vmem()] * 3,
        out_shape=[jax.ShapeDtypeStruct((1, 1), jnp.float32),
                   jax.ShapeDtypeStruct((e_all, 1), jnp.float32),
                   jax.ShapeDtypeStruct((e_pos, fout), jnp.float32)],
    )(feat, attr, nnw3, nnb2, src, tgt_row, src_all, tgt_all,
      root_w, root_b, lin1_w, lin1_b, lin2_w, lin2_b)


def kernel(conv1_w, conv1_b, conv2_w, conv2_b, conv3_w, conv3_b,
           pool1_w_rel, pool1_w_root, pool1_b_rel,
           pool2_w_rel, pool2_w_root, pool2_b_rel,
           pool3_w_rel, pool3_w_root, pool3_b_rel,
           nn_w, nn_b, conv4_root, conv4_b, lin1_w, lin1_b,
           lin2_w, lin2_b, lin3_w, lin3_b,
           x_all, aw_all, au_all,
           ddi_edge_index, neg_edge_index, ddi_edge_attr, neg_edge_attr):
    modular_feature = _encoder(
        x_all, aw_all, au_all,
        (conv1_w, conv2_w, conv3_w), (conv1_b, conv2_b, conv3_b),
        (pool1_w_rel, pool1_w_root, pool2_w_rel, pool2_w_root,
         pool3_w_rel, pool3_w_root),
        (pool1_b_rel, pool2_b_rel, pool3_b_rel), 0.5)

    mn, fin = modular_feature.shape
    fout = conv4_root.shape[1]
    c = ddi_edge_attr.shape[1]
    e_pos = ddi_edge_index.shape[1]
    e_neg = neg_edge_index.shape[1]

    nnw3 = nn_w.reshape(c, fin, fout)
    nnb2 = nn_b.reshape(fin, fout)

    src = ddi_edge_index[0].astype(jnp.int32).reshape(e_pos, 1)
    tgt_row = ddi_edge_index[1].astype(jnp.int32).reshape(1, e_pos)
    src_all = jnp.concatenate([ddi_edge_index[0], neg_edge_index[0]]
                              ).astype(jnp.int32).reshape(e_pos + e_neg, 1)
    tgt_all = jnp.concatenate([ddi_edge_index[1], neg_edge_index[1]]
                              ).astype(jnp.int32).reshape(e_pos + e_neg, 1)

    loss, scores, pos_feat_x = _ddi_head(
        modular_feature, ddi_edge_attr, nnw3, nnb2, src, tgt_row, src_all, tgt_all,
        conv4_root, conv4_b.reshape(1, -1),
        lin1_w, lin1_b.reshape(1, -1), lin2_w, lin2_b.reshape(1, -1),
        e_pos=e_pos, e_neg=e_neg)

    return loss[0, 0], scores[:e_pos, 0], scores[e_pos:, 0], pos_feat_x


# speedup vs baseline: 1.8877x; 1.8877x over previous
import functools
import math

import jax
import jax.numpy as jnp
from jax.experimental import pallas as pl
from jax.experimental.pallas import tpu as pltpu


def _full_spec(shape):
    zeros = (0,) * len(shape)
    return pl.BlockSpec(tuple(shape), lambda b: zeros)


def _relayout_kernel(a_ref, u_ref, oa_ref, ou_ref, *, n, cb):
    for src, dst in ((a_ref, oa_ref), (u_ref, ou_ref)):
        two = src[...].reshape(n * n, cb)
        dst[...] = jnp.transpose(two, (1, 0)).reshape(cb, n, n)


def _relayout(aw_all, au_all):
    m, n, _ = aw_all.shape
    cb = 128 if m % 128 == 0 else m
    va = jnp.transpose(aw_all, (1, 2, 0))
    vu = jnp.transpose(au_all, (1, 2, 0))
    return pl.pallas_call(
        functools.partial(_relayout_kernel, n=n, cb=cb),
        grid=(m // cb,),
        in_specs=[pl.BlockSpec((n, n, cb), lambda b: (0, 0, b)),
                  pl.BlockSpec((n, n, cb), lambda b: (0, 0, b))],
        out_specs=[pl.BlockSpec((cb, n, n), lambda b: (b, 0, 0)),
                   pl.BlockSpec((cb, n, n), lambda b: (b, 0, 0))],
        out_shape=[jax.ShapeDtypeStruct((m, n, n), jnp.float32),
                   jax.ShapeDtypeStruct((m, n, n), jnp.float32)],
        compiler_params=pltpu.CompilerParams(
            dimension_semantics=("arbitrary",)),
    )(va, vu)


def _encoder_kernel(x_ref, aw_ref, au_ref,
                    w1_ref, b1_ref, w2_ref, b2_ref, w3_ref, b3_ref,
                    whh1_ref, pb1_ref, whh2_ref, pb2_ref, whh3_ref, pb3_ref,
                    out_ref, *, g, n0, k1, k2, k3):
    f32 = jnp.float32

    def level(aw2, au2, xs, w_ref, b_ref, whh_ref, brel_ref,
              n, k, want_adj):
        w = w_ref[...]
        b_row = b_ref[...]
        xw = jnp.dot(xs, w, preferred_element_type=f32)
        d = jax.lax.rsqrt(jnp.sum(aw2, axis=1, keepdims=True) + 1.0)
        dxw = d * xw
        agg = jnp.concatenate(
            [jnp.dot(aw2[i * n:(i + 1) * n, :], dxw[i * n:(i + 1) * n, :],
                     preferred_element_type=f32) for i in range(g)], axis=0)
        h = jnp.maximum(d * agg + (d * d) * xw + b_row, 0.0)

        hwr = jnp.dot(h, whh_ref[...], preferred_element_type=f32)
        hw = hwr[:, 0:1]
        brel = brel_ref[...]
        s = jnp.concatenate(
            [jnp.tanh(jnp.dot(au2[i * n:(i + 1) * n, :],
                              hw[i * n:(i + 1) * n, :],
                              preferred_element_type=f32)
                      + hwr[i * n:(i + 1) * n, 1:2] + brel)
             for i in range(g)], axis=0)
        hs = h * s

        sub = jax.lax.broadcasted_iota(jnp.int32, (n, 1), 0)
        lane = jax.lax.broadcasted_iota(jnp.int32, (1, n), 1)
        eye = sub == lane
        ksub = jax.lax.broadcasted_iota(jnp.int32, (k, 1), 0)
        nh = xw.shape[1]
        new_aw, new_au, xs_next, ros = [], [], [], []
        for i in range(g):
            si = s[i * n:(i + 1) * n, :]
            s_row = jnp.sum(jnp.where(eye, si, 0.0), axis=0,
                            keepdims=True)
            dom = (si > s_row) | ((si == s_row) & (sub < lane))
            rank = jnp.sum(dom.astype(jnp.int32), axis=0,
                           keepdims=True)
            p = (rank == ksub).astype(f32)
            if want_adj:
                cat = jnp.concatenate(
                    [hs[i * n:(i + 1) * n, :], aw2[i * n:(i + 1) * n, :],
                     au2[i * n:(i + 1) * n, :]], axis=1)
                pooled = jnp.dot(p, cat, preferred_element_type=f32)
                xp = pooled[:, :nh]
                both = jax.lax.dot_general(
                    jnp.concatenate([pooled[:, nh:nh + n],
                                     pooled[:, nh + n:]], axis=0), p,
                    (((1,), (1,)), ((), ())),
                    preferred_element_type=f32)
                new_aw.append(both[:k, :])
                new_au.append(both[k:, :])
            else:
                xp = jnp.dot(p, hs[i * n:(i + 1) * n, :],
                             preferred_element_type=f32)
            xs_next.append(xp)
            ros.append((jnp.max(xp, axis=0, keepdims=True),
                        jnp.mean(xp, axis=0, keepdims=True)))
        naw = jnp.concatenate(new_aw, axis=0) if want_adj else None
        nau = jnp.concatenate(new_au, axis=0) if want_adj else None
        return naw, nau, jnp.concatenate(xs_next, axis=0), ros

    aw1 = aw_ref[...].reshape(g * n0, n0)
    au1 = au_ref[...].reshape(g * n0, n0)
    x0 = x_ref[...].reshape(g * n0, x_ref.shape[2])

    aw2, au2, xp1, ro1 = level(aw1, au1, x0, w1_ref, b1_ref,
                               whh1_ref, pb1_ref, n0, k1, True)
    aw3, au3, xp2, ro2 = level(aw2, au2, xp1, w2_ref, b2_ref,
                               whh2_ref, pb2_ref, k1, k2, True)
    _, _, _, ro3 = level(aw3, au3, xp2, w3_ref, b3_ref,
                         whh3_ref, pb3_ref, k2, k3, False)

    rows = [jnp.concatenate([ro1[i][0], ro1[i][1], ro2[i][0], ro2[i][1],
                             ro3[i][0], ro3[i][1]], axis=1)
            for i in range(g)]
    out_ref[...] = jnp.concatenate(rows, axis=0)


def _encoder(x_all, aw_all, au_all, conv_ws, conv_bs, pool_ws, pool_bs, ratio):
    mb, n0, f = x_all.shape
    nh = conv_ws[0].shape[1]
    k1 = int(math.ceil(ratio * n0))
    k2 = int(math.ceil(ratio * k1))
    k3 = int(math.ceil(ratio * k2))
    g = 32 if mb % 32 == 0 else 1

    w1, w2, w3 = conv_ws
    b1, b2, b3 = (b.reshape(1, nh) for b in conv_bs)
    pr1, po1, pr2, po2, pr3, po3 = pool_ws
    whh1 = jnp.concatenate([pr1, po1], axis=1)
    whh2 = jnp.concatenate([pr2, po2], axis=1)
    whh3 = jnp.concatenate([pr3, po3], axis=1)
    pb1, pb2, pb3 = (b.reshape(1, 1) for b in pool_bs)

    aw_all, au_all = _relayout(aw_all, au_all)

    out = pl.pallas_call(
        functools.partial(_encoder_kernel, g=g, n0=n0, k1=k1, k2=k2, k3=k3),
        grid=(mb // g,),
        in_specs=[
            pl.BlockSpec((g, n0, f), lambda b: (b, 0, 0)),
            pl.BlockSpec((g, n0, n0), lambda b: (b, 0, 0)),
            pl.BlockSpec((g, n0, n0), lambda b: (b, 0, 0)),
            _full_spec((f, nh)), _full_spec((1, nh)),
            _full_spec((nh, nh)), _full_spec((1, nh)),
            _full_spec((nh, nh)), _full_spec((1, nh)),
            _full_spec((nh, 2)), _full_spec((1, 1)),
            _full_spec((nh, 2)), _full_spec((1, 1)),
            _full_spec((nh, 2)), _full_spec((1, 1)),
        ],
        out_specs=pl.BlockSpec((g, 6 * nh), lambda b: (b, 0)),
        out_shape=jax.ShapeDtypeStruct((mb, 6 * nh), jnp.float32),
        compiler_params=pltpu.CompilerParams(
            dimension_semantics=("parallel",)),
    )(x_all, aw_all, au_all, w1, b1, w2, b2, w3, b3,
      whh1, pb1, whh2, pb2, whh3, pb3)
    return out


def _ddi_kernel(feat_ref, attr_ref, nnw_ref, nnb_ref,
                src_ref, tgt_ref, asrc_ref, atgt_ref,
                root_w_ref, root_b_ref, lin1_w_ref, lin1_b_ref,
                lin2_w_ref, lin2_b_ref,
                loss_ref, score_ref, posx_ref, *, e_pos, e_neg, n_nodes, n_chan):
    f32 = jnp.float32
    feat = feat_ref[...]
    attr = attr_ref[...]
    src = src_ref[...]
    tgt = tgt_ref[...]
    asrc = asrc_ref[...]
    atgt = atgt_ref[...]

    node_lane = jax.lax.broadcasted_iota(jnp.int32, (1, n_nodes), 1)
    node_sub = jax.lax.broadcasted_iota(jnp.int32, (n_nodes, 1), 0)

    gsrc = (src == node_lane).astype(f32)
    xs = jnp.dot(gsrc, feat, preferred_element_type=f32)

    msg = jnp.dot(xs, nnb_ref[...], preferred_element_type=f32)
    for c in range(n_chan):
        msg = msg + attr[:, c:c + 1] * jnp.dot(xs, nnw_ref[c],
                                               preferred_element_type=f32)

    scat = (node_sub == tgt).astype(f32)
    agg = jnp.dot(scat, msg, preferred_element_type=f32)

    x_ddi = jnp.maximum(jnp.dot(feat, root_w_ref[...],
                                preferred_element_type=f32)
                        + agg + root_b_ref[...], 0.0)

    gsrc_all = (asrc == node_lane).astype(f32)
    gtgt_all = (atgt == node_lane).astype(f32)
    src_feat = jnp.dot(gsrc_all, x_ddi, preferred_element_type=f32)
    tgt_feat = jnp.dot(gtgt_all, x_ddi, preferred_element_type=f32)
    fx = jnp.dot(src_feat, lin1_w_ref[...],
                 preferred_element_type=f32) + lin1_b_ref[...]
    fy = jnp.dot(tgt_feat, lin2_w_ref[...],
                 preferred_element_type=f32) + lin2_b_ref[...]

    scores = jnp.sum(fx * fy, axis=1, keepdims=True)
    score_ref[...] = scores
    posx_ref[...] = fx[:e_pos, :]

    def sp(v):
        return jnp.maximum(v, 0.0) + jnp.log(1.0 + jnp.exp(-jnp.abs(v)))

    row = jax.lax.broadcasted_iota(jnp.int32, scores.shape, 0)
    is_pos = row < e_pos
    loss_pos = jnp.sum(jnp.where(is_pos, sp(-scores), 0.0),
                       axis=0, keepdims=True) / e_pos
    loss_neg = jnp.sum(jnp.where(is_pos, 0.0, sp(scores)),
                       axis=0, keepdims=True) / e_neg
    loss_ref[...] = loss_pos + loss_neg


def _vmem():
    return pl.BlockSpec(memory_space=pltpu.MemorySpace.VMEM)


def _ddi_head(feat, attr, nnw3, nnb2, src, tgt_row, src_all, tgt_all,
              root_w, root_b, lin1_w, lin1_b, lin2_w, lin2_b,
              *, e_pos, e_neg):
    m, _ = feat.shape
    fout = root_w.shape[1]
    c = attr.shape[1]
    e_all = src_all.shape[0]
    return pl.pallas_call(
        functools.partial(_ddi_kernel, e_pos=e_pos, e_neg=e_neg,
                          n_nodes=m, n_chan=c),
        in_specs=[_vmem()] * 14,
        out_specs=[_vmem()] * 3,
        out_shape=[jax.ShapeDtypeStruct((1, 1), jnp.float32),
                   jax.ShapeDtypeStruct((e_all, 1), jnp.float32),
                   jax.ShapeDtypeStruct((e_pos, fout), jnp.float32)],
    )(feat, attr, nnw3, nnb2, src, tgt_row, src_all, tgt_all,
      root_w, root_b, lin1_w, lin1_b, lin2_w, lin2_b)


def kernel(conv1_w, conv1_b, conv2_w, conv2_b, conv3_w, conv3_b,
           pool1_w_rel, pool1_w_root, pool1_b_rel,
           pool2_w_rel, pool2_w_root, pool2_b_rel,
           pool3_w_rel, pool3_w_root, pool3_b_rel,
           nn_w, nn_b, conv4_root, conv4_b, lin1_w, lin1_b,
           lin2_w, lin2_b, lin3_w, lin3_b,
           x_all, aw_all, au_all,
           ddi_edge_index, neg_edge_index, ddi_edge_attr, neg_edge_attr):
    modular_feature = _encoder(
        x_all, aw_all, au_all,
        (conv1_w, conv2_w, conv3_w), (conv1_b, conv2_b, conv3_b),
        (pool1_w_rel, pool1_w_root, pool2_w_rel, pool2_w_root,
         pool3_w_rel, pool3_w_root),
        (pool1_b_rel, pool2_b_rel, pool3_b_rel), 0.5)

    mn, fin = modular_feature.shape
    fout = conv4_root.shape[1]
    c = ddi_edge_attr.shape[1]
    e_pos = ddi_edge_index.shape[1]
    e_neg = neg_edge_index.shape[1]

    nnw3 = nn_w.reshape(c, fin, fout)
    nnb2 = nn_b.reshape(fin, fout)

    src = ddi_edge_index[0].astype(jnp.int32).reshape(e_pos, 1)
    tgt_row = ddi_edge_index[1].astype(jnp.int32).reshape(1, e_pos)
    src_all = jnp.concatenate([ddi_edge_index[0], neg_edge_index[0]]
                              ).astype(jnp.int32).reshape(e_pos + e_neg, 1)
    tgt_all = jnp.concatenate([ddi_edge_index[1], neg_edge_index[1]]
                              ).astype(jnp.int32).reshape(e_pos + e_neg, 1)

    loss, scores, pos_feat_x = _ddi_head(
        modular_feature, ddi_edge_attr, nnw3, nnb2, src, tgt_row, src_all, tgt_all,
        conv4_root, conv4_b.reshape(1, -1),
        lin1_w, lin1_b.reshape(1, -1), lin2_w, lin2_b.reshape(1, -1),
        e_pos=e_pos, e_neg=e_neg)

    return loss[0, 0], scores[:e_pos, 0], scores[e_pos:, 0], pos_feat_x
